# Initial kernel scaffold; baseline (speedup 1.0000x reference)
import jax
import jax.numpy as jnp
from jax.experimental import pallas as pl


def kernel(x, table):
    raise NotImplementedError("write your pallas kernel here")


# speedup vs baseline: 3.9535x; 3.9535x over previous
import functools

import jax
import jax.numpy as jnp
from jax import lax
from jax.experimental import pallas as pl
from jax.experimental.pallas import tpu as pltpu
from jax.experimental.pallas import tpu_sc as plsc

DIM = 64


@functools.lru_cache(maxsize=None)
def _make_gather(B: int, C: int):
    info = plsc.get_sparse_core_info()
    NC, NS = info.num_cores, info.num_subcores
    NW = NC * NS
    n_per_w = B // NW
    steps = n_per_w // C
    assert steps * C == n_per_w and n_per_w * NW == B
    mesh = plsc.VectorSubcoreMesh(core_axis_name="c", subcore_axis_name="s")

    @functools.partial(
        pl.kernel,
        mesh=mesh,
        out_type=jax.ShapeDtypeStruct((B, DIM), jnp.float32),
        compiler_params=pltpu.CompilerParams(use_tc_tiling_on_sc=False),
        scratch_types=[
            pltpu.VMEM((C,), jnp.int32),
            pltpu.VMEM((C, DIM), jnp.float32),
            pltpu.SemaphoreType.DMA,
        ],
    )
    def gather_kernel(idx_hbm, table_hbm, out_hbm, idx_v, rows_v, sem):
        wid = lax.axis_index("s") * NC + lax.axis_index("c")
        base = wid * n_per_w

        def body(i, carry):
            off = base + i * C
            pltpu.sync_copy(idx_hbm.at[pl.ds(off, C)], idx_v)
            pltpu.async_copy(table_hbm.at[idx_v], rows_v, sem).wait()
            pltpu.sync_copy(rows_v, out_hbm.at[pl.ds(off, C)])
            return carry

        lax.fori_loop(0, steps, body, 0)

    return gather_kernel


def kernel(x, table):
    rows, cols = x.shape
    B = rows * cols
    xf = x.reshape(B).astype(jnp.int32)
    out = _make_gather(B, 512)(xf, table)
    return out.reshape(rows, cols, DIM)


# speedup vs baseline: 4.2621x; 1.0781x over previous
import functools

import jax
import jax.numpy as jnp
from jax import lax
from jax.experimental import pallas as pl
from jax.experimental.pallas import tpu as pltpu
from jax.experimental.pallas import tpu_sc as plsc

DIM = 64


@functools.lru_cache(maxsize=None)
def _make_gather(B: int, C: int):
    info = plsc.get_sparse_core_info()
    NC, NS = info.num_cores, info.num_subcores
    NW = NC * NS
    n_per_w = B // NW
    steps = n_per_w // C
    assert steps * C == n_per_w and n_per_w * NW == B and steps % 2 == 0
    mesh = plsc.VectorSubcoreMesh(core_axis_name="c", subcore_axis_name="s")

    @functools.partial(
        pl.kernel,
        mesh=mesh,
        out_type=jax.ShapeDtypeStruct((B, DIM), jnp.float32),
        compiler_params=pltpu.CompilerParams(use_tc_tiling_on_sc=False),
        scratch_types=[
            pltpu.VMEM((n_per_w,), jnp.int32),
            pltpu.VMEM((2, C, DIM), jnp.float32),
            pltpu.SemaphoreType.DMA,
            pltpu.SemaphoreType.DMA,
            pltpu.SemaphoreType.DMA,
            pltpu.SemaphoreType.DMA,
        ],
    )
    def gather_kernel(idx_hbm, table_hbm, out_hbm, idx_v, rows_v, g0, g1, w0, w1):
        gsems = (g0, g1)
        wsems = (w0, w1)
        wid = lax.axis_index("s") * NC + lax.axis_index("c")
        base = wid * n_per_w

        pltpu.sync_copy(idx_hbm.at[pl.ds(base, n_per_w)], idx_v)

        def gather_start(cur, b):
            pltpu.async_copy(
                table_hbm.at[idx_v.at[pl.ds(cur * C, C)]], rows_v.at[b], gsems[b]
            )

        def gather_wait(b):
            pltpu.make_async_copy(
                table_hbm.at[idx_v.at[pl.ds(0, C)]], rows_v.at[b], gsems[b]
            ).wait()

        def wb_start(cur, b):
            pltpu.async_copy(
                rows_v.at[b], out_hbm.at[pl.ds(base + cur * C, C)], wsems[b]
            )

        def wb_wait(b):
            pltpu.make_async_copy(
                rows_v.at[b], out_hbm.at[pl.ds(base, C)], wsems[b]
            ).wait()

        gather_start(0, 0)

        def body(g, carry):
            for b in range(2):
                cur = g + b

                @pl.when(cur >= 1)
                def _():
                    wb_wait(1 - b)

                @pl.when(cur + 1 < steps)
                def _():
                    gather_start(cur + 1, 1 - b)

                gather_wait(b)
                wb_start(cur, b)
            return carry

        lax.fori_loop(0, steps // 2, lambda i, c: body(i * 2, c), 0)
        wb_wait((steps - 1) % 2)

    return gather_kernel


def kernel(x, table):
    rows, cols = x.shape
    B = rows * cols
    xf = x.reshape(B).astype(jnp.int32)
    out = _make_gather(B, 512)(xf, table)
    return out.reshape(rows, cols, DIM)


# speedup vs baseline: 4.2652x; 1.0007x over previous
import functools

import jax
import jax.numpy as jnp
from jax import lax
from jax.experimental import pallas as pl
from jax.experimental.pallas import tpu as pltpu
from jax.experimental.pallas import tpu_sc as plsc

DIM = 64


@functools.lru_cache(maxsize=None)
def _make_gather(B: int, C: int):
    info = plsc.get_sparse_core_info()
    NC, NS = info.num_cores, info.num_subcores
    NW = NC * NS
    n_per_w = B // NW
    steps = n_per_w // C
    assert steps * C == n_per_w and n_per_w * NW == B and steps % 2 == 0
    mesh = plsc.VectorSubcoreMesh(core_axis_name="c", subcore_axis_name="s")

    @functools.partial(
        pl.kernel,
        mesh=mesh,
        out_type=jax.ShapeDtypeStruct((B, DIM), jnp.float32),
        compiler_params=pltpu.CompilerParams(use_tc_tiling_on_sc=False),
        scratch_types=[
            pltpu.VMEM((n_per_w,), jnp.int32),
            pltpu.VMEM((2, C, DIM), jnp.float32),
            pltpu.SemaphoreType.DMA,
            pltpu.SemaphoreType.DMA,
            pltpu.SemaphoreType.DMA,
            pltpu.SemaphoreType.DMA,
        ],
    )
    def gather_kernel(idx_hbm, table_hbm, out_hbm, idx_v, rows_v, g0, g1, w0, w1):
        gsems = (g0, g1)
        wsems = (w0, w1)
        wid = lax.axis_index("s") * NC + lax.axis_index("c")
        base = wid * n_per_w

        pltpu.sync_copy(idx_hbm.at[pl.ds(base, n_per_w)], idx_v)

        def gather_start(cur, b):
            pltpu.async_copy(
                table_hbm.at[idx_v.at[pl.ds(cur * C, C)]], rows_v.at[b], gsems[b]
            )

        def gather_wait(b):
            pltpu.make_async_copy(
                table_hbm.at[idx_v.at[pl.ds(0, C)]], rows_v.at[b], gsems[b]
            ).wait()

        def wb_start(cur, b):
            pltpu.async_copy(
                rows_v.at[b], out_hbm.at[pl.ds(base + cur * C, C)], wsems[b]
            )

        def wb_wait(b):
            pltpu.make_async_copy(
                rows_v.at[b], out_hbm.at[pl.ds(base, C)], wsems[b]
            ).wait()

        gather_start(0, 0)

        def body(g, carry):
            for b in range(2):
                cur = g + b

                @pl.when(cur >= 1)
                def _():
                    wb_wait(1 - b)

                @pl.when(cur + 1 < steps)
                def _():
                    gather_start(cur + 1, 1 - b)

                gather_wait(b)
                wb_start(cur, b)
            return carry

        lax.fori_loop(0, steps // 2, lambda i, c: body(i * 2, c), 0)
        wb_wait((steps - 1) % 2)

    return gather_kernel


def kernel(x, table):
    rows, cols = x.shape
    B = rows * cols
    V = table.shape[0]
    xf = x.reshape(B).astype(jnp.int32)
    t2 = lax.optimization_barrier(table.reshape(V // 2, 2 * DIM))
    t3 = t2.reshape(V, DIM)
    out = _make_gather(B, 512)(xf, t3)
    return out.reshape(rows, cols, DIM)
